# baseline (speedup 1.0000x reference)
import jax
import jax.numpy as jnp
from jax import lax
from jax.experimental import pallas as pl
from jax.experimental.pallas import tpu as pltpu
from jax.experimental.pallas import tpu_sc as plsc

VOCAB_ROWS = 1_000_000
BATCH = 16384
EMBED = 64
WIDE = 2 * EMBED
NUM_CORES = 2
NUM_SUBCORES = 16
NUM_TILES = NUM_CORES * NUM_SUBCORES
B_PER_TILE = BATCH // NUM_TILES


def kernel(inputs, W):
    idx = inputs.reshape((BATCH,))
    Wp = jnp.pad(W, ((0, 0), (0, WIDE - EMBED)))

    mesh = plsc.VectorSubcoreMesh(core_axis_name="c", subcore_axis_name="s")

    @pl.kernel(
        out_type=jax.ShapeDtypeStruct((BATCH, WIDE), W.dtype),
        mesh=mesh,
        scratch_types=[
            pltpu.VMEM((B_PER_TILE,), jnp.int32),
            pltpu.VMEM((B_PER_TILE, WIDE), jnp.float32),
            pltpu.SemaphoreType.DMA,
        ],
    )
    def gather_kernel(table_hbm, idx_hbm, out_hbm, idx_v, rows_v, sem):
        wid = lax.axis_index("s") * NUM_CORES + lax.axis_index("c")
        base = wid * B_PER_TILE
        pltpu.sync_copy(idx_hbm.at[pl.ds(base, B_PER_TILE)], idx_v)
        pltpu.async_copy(table_hbm.at[idx_v], rows_v, sem).wait()
        pltpu.sync_copy(rows_v, out_hbm.at[pl.ds(base, B_PER_TILE)])

    big = gather_kernel(Wp, idx)
    return big[:, :EMBED]


# speedup vs baseline: 1.4749x; 1.4749x over previous
import jax
import jax.numpy as jnp
from jax import lax
from jax.experimental import pallas as pl
from jax.experimental.pallas import tpu as pltpu
from jax.experimental.pallas import tpu_sc as plsc

BATCH = 16384
EMBED = 64
NUM_CORES = 2
NUM_SUBCORES = 16
NUM_TILES = NUM_CORES * NUM_SUBCORES
B_PER_TILE = BATCH // NUM_TILES
LANES = 16
N_CHUNKS = B_PER_TILE // LANES
SPLIT = 256
STAGED = B_PER_TILE - SPLIT


def kernel(inputs, W):
    idx = inputs.reshape((BATCH,))

    mesh = plsc.VectorSubcoreMesh(core_axis_name="c", subcore_axis_name="s")

    @pl.kernel(
        out_type=jax.ShapeDtypeStruct((BATCH, EMBED), W.dtype),
        mesh=mesh,
        scratch_types=[
            pltpu.VMEM((B_PER_TILE,), jnp.int32),
            pltpu.VMEM((STAGED, EMBED), jnp.float32),
            pltpu.VMEM_SHARED((NUM_SUBCORES * SPLIT, EMBED), jnp.float32),
            pltpu.SemaphoreType.DMA,
            pltpu.SemaphoreType.DMA,
            pltpu.SemaphoreType.DMA,
        ],
        compiler_params=pltpu.CompilerParams(needs_layout_passes=False),
    )
    def gather_kernel(table_hbm, idx_hbm, out_hbm, idx_v, rows_v, shared_v,
                      sem_i, sem_t, sem_s):
        c_id = lax.axis_index("c")
        s_id = lax.axis_index("s")
        wid = s_id * NUM_CORES + c_id
        base = wid * B_PER_TILE
        sbase = s_id * SPLIT
        pltpu.async_copy(idx_hbm.at[pl.ds(base, B_PER_TILE)], idx_v, sem_i).wait()

        lane = lax.broadcasted_iota(jnp.int32, (LANES,), 0)

        @pl.loop(0, N_CHUNKS)
        def _(c):
            chunk = idx_v[pl.ds(c * LANES, LANES)]
            for j in range(LANES):
                i = jnp.sum(jnp.where(lane == j, chunk, 0))
                b = c * LANES + j

                @pl.when(b < SPLIT)
                def _():
                    pltpu.make_async_copy(
                        table_hbm.at[pl.ds(i, 1)],
                        shared_v.at[pl.ds(sbase + b, 1)],
                        sem_s,
                    ).start()

                @pl.when(b >= SPLIT)
                def _():
                    pltpu.make_async_copy(
                        table_hbm.at[pl.ds(i, 1)],
                        rows_v.at[pl.ds(b - SPLIT, 1)],
                        sem_t,
                    ).start()

        @pl.loop(0, SPLIT)
        def _(b):
            pltpu.make_async_copy(
                table_hbm.at[pl.ds(0, 1)],
                shared_v.at[pl.ds(sbase + b, 1)],
                sem_s,
            ).wait()

        @pl.loop(0, STAGED)
        def _(b):
            pltpu.make_async_copy(
                table_hbm.at[pl.ds(0, 1)],
                rows_v.at[pl.ds(b, 1)],
                sem_t,
            ).wait()

        pltpu.sync_copy(shared_v.at[pl.ds(sbase, SPLIT)],
                        out_hbm.at[pl.ds(base, SPLIT)])
        pltpu.sync_copy(rows_v, out_hbm.at[pl.ds(base + SPLIT, STAGED)])

    return gather_kernel(W, idx)
